# baseline (speedup 1.0000x reference)
import functools

import jax
import jax.numpy as jnp
from jax import lax
from jax.experimental import pallas as pl
from jax.experimental.pallas import tpu as pltpu
from jax.experimental.pallas import tpu_sc as plsc

VOCAB = 49408
HIDDEN = 768
SEQ = 77
BATCH = 1024

NC = 2
NS = 16
LANES = 16
NW = NC * NS
BPW = BATCH // NW
KV = HIDDEN // LANES
NSLOT = 4


def _body(ids_hbm, table_hbm, pos_hbm, out_hbm, ids_v,
          rows0, rows1, rows2, rows3, prow0, prow1, prow2, prow3,
          gsem0, gsem1, gsem2, gsem3, ssem0, ssem1, ssem2, ssem3,
          psem0, psem1, psem2, psem3):
  rows = [rows0, rows1, rows2, rows3]
  prow = [prow0, prow1, prow2, prow3]
  gsem = [gsem0, gsem1, gsem2, gsem3]
  ssem = [ssem0, ssem1, ssem2, ssem3]
  psem = [psem0, psem1, psem2, psem3]

  wid = lax.axis_index("s") * NC + lax.axis_index("c")

  pltpu.sync_copy(ids_hbm.at[wid], ids_v)

  def launch(s, b):
    pltpu.async_copy(table_hbm.at[ids_v.at[s]], rows[b], gsem[b])
    pltpu.async_copy(pos_hbm.at[pl.ds(s * HIDDEN, HIDDEN)], prow[b], psem[b])

  def sc_dst(s):
    return out_hbm.at[pl.ds(s * BATCH + wid * BPW, BPW)]

  def add_pos(b):
    def kbody(k, _):
      pv = prow[b][pl.ds(k * LANES, LANES)]
      def jbody(j, _):
        r = rows[b][j, pl.ds(k * LANES, LANES)]
        rows[b][j, pl.ds(k * LANES, LANES)] = r + pv
        return 0
      return lax.fori_loop(0, BPW, jbody, 0, unroll=8)
    lax.fori_loop(0, KV, kbody, 0)

  def visit(s, b):
    b2 = (b - 2) % NSLOT
    pltpu.make_async_copy(table_hbm.at[ids_v.at[s]], rows[b], gsem[b]).wait()
    pltpu.make_async_copy(
        pos_hbm.at[pl.ds(s * HIDDEN, HIDDEN)], prow[b], psem[b]).wait()
    add_pos(b)
    pltpu.async_copy(rows[b], sc_dst(s), ssem[b])
    @pl.when(s >= 2)
    def _():
      pltpu.make_async_copy(rows[b2], sc_dst(s - 2), ssem[b2]).wait()
    @pl.when(s + 2 < SEQ)
    def _():
      launch(s + 2, b2)

  launch(0, 0)
  launch(1, 1)

  def loop_body(t, _):
    for b in range(NSLOT):
      visit(NSLOT * t + b, b)
    return 0
  lax.fori_loop(0, SEQ // NSLOT, loop_body, 0)
  visit(SEQ - 1, (SEQ - 1) % NSLOT)

  pltpu.make_async_copy(rows[3], sc_dst(SEQ - 2), ssem[3]).wait()
  pltpu.make_async_copy(rows[0], sc_dst(SEQ - 1), ssem[0]).wait()


@functools.partial(jax.jit, donate_argnums=())
def _embed(ids_w, table, pos):
  mesh = plsc.VectorSubcoreMesh(
      core_axis_name="c", subcore_axis_name="s",
      num_cores=NC, num_subcores=NS)
  run = pl.kernel(
      _body,
      out_type=jax.ShapeDtypeStruct((BATCH * SEQ, HIDDEN), jnp.float32),
      mesh=mesh,
      scratch_types=(
          [pltpu.VMEM((SEQ, BPW), jnp.int32)]
          + [pltpu.VMEM((BPW, HIDDEN), jnp.float32)] * NSLOT
          + [pltpu.VMEM((HIDDEN,), jnp.float32)] * NSLOT
          + [pltpu.SemaphoreType.DMA] * (3 * NSLOT)
      ),
  )
  return run(ids_w, table, pos)


def kernel(input_ids, token_embedding, position_embedding):
  ids32 = input_ids.astype(jnp.int32)
  ids_w = ids32.reshape(NW, BPW, SEQ).transpose(0, 2, 1)
  pos_flat = position_embedding.reshape(SEQ * HIDDEN)
  out = _embed(ids_w, token_embedding, pos_flat)
  return out.reshape(SEQ, BATCH, HIDDEN).transpose(1, 0, 2)


# speedup vs baseline: 1.0846x; 1.0846x over previous
import functools

import jax
import jax.numpy as jnp
from jax import lax
from jax.experimental import pallas as pl
from jax.experimental.pallas import tpu as pltpu
from jax.experimental.pallas import tpu_sc as plsc

VOCAB = 49408
HIDDEN = 768
SEQ = 77
BATCH = 1024

NC = 2
NS = 16
LANES = 16
NW = NC * NS
BPW = BATCH // NW
KV = HIDDEN // LANES
NSLOT = 4


def _body(ids_hbm, table_hbm, pos_hbm, flags_hbm, out_hbm, ids_v, flags_v,
          rows0, rows1, rows2, rows3, prow0, prow1, prow2, prow3,
          gsem0, gsem1, gsem2, gsem3, ssem0, ssem1, ssem2, ssem3,
          psem0, psem1, psem2, psem3):
  rows = [rows0, rows1, rows2, rows3]
  prow = [prow0, prow1, prow2, prow3]
  gsem = [gsem0, gsem1, gsem2, gsem3]
  ssem = [ssem0, ssem1, ssem2, ssem3]
  psem = [psem0, psem1, psem2, psem3]

  wid = lax.axis_index("s") * NC + lax.axis_index("c")

  pltpu.sync_copy(ids_hbm.at[wid], ids_v)
  pltpu.sync_copy(flags_hbm, flags_v)

  def flag(s):
    return flags_v[s, pl.ds(0, LANES)][0]

  def launch(s, b):
    pltpu.async_copy(table_hbm.at[ids_v.at[s]], rows[b], gsem[b])
    @pl.when(flag(s) != 0)
    def _():
      pltpu.async_copy(pos_hbm.at[pl.ds(s * HIDDEN, HIDDEN)], prow[b], psem[b])

  def sc_dst(s):
    return out_hbm.at[pl.ds(s * BATCH + wid * BPW, BPW)]

  def add_pos(s, b):
    @pl.when(flag(s) != 0)
    def _():
      pltpu.make_async_copy(
          pos_hbm.at[pl.ds(s * HIDDEN, HIDDEN)], prow[b], psem[b]).wait()
      def kbody(k, _):
        pv = prow[b][pl.ds(k * LANES, LANES)]
        def jbody(j, _):
          r = rows[b][j, pl.ds(k * LANES, LANES)]
          rows[b][j, pl.ds(k * LANES, LANES)] = r + pv
          return 0
        return lax.fori_loop(0, BPW, jbody, 0, unroll=8)
      lax.fori_loop(0, KV, kbody, 0)

  def visit(s, b):
    b2 = (b - 2) % NSLOT
    pltpu.make_async_copy(table_hbm.at[ids_v.at[s]], rows[b], gsem[b]).wait()
    add_pos(s, b)
    pltpu.async_copy(rows[b], sc_dst(s), ssem[b])
    @pl.when(s >= 2)
    def _():
      pltpu.make_async_copy(rows[b2], sc_dst(s - 2), ssem[b2]).wait()
    @pl.when(s + 2 < SEQ)
    def _():
      launch(s + 2, b2)

  launch(0, 0)
  launch(1, 1)

  def loop_body(t, _):
    for b in range(NSLOT):
      visit(NSLOT * t + b, b)
    return 0
  lax.fori_loop(0, SEQ // NSLOT, loop_body, 0)
  visit(SEQ - 1, (SEQ - 1) % NSLOT)

  pltpu.make_async_copy(rows[3], sc_dst(SEQ - 2), ssem[3]).wait()
  pltpu.make_async_copy(rows[0], sc_dst(SEQ - 1), ssem[0]).wait()


@functools.partial(jax.jit, donate_argnums=())
def _embed(ids_w, table, pos, flags):
  mesh = plsc.VectorSubcoreMesh(
      core_axis_name="c", subcore_axis_name="s",
      num_cores=NC, num_subcores=NS)
  run = pl.kernel(
      _body,
      out_type=jax.ShapeDtypeStruct((BATCH * SEQ, HIDDEN), jnp.float32),
      mesh=mesh,
      scratch_types=(
          [pltpu.VMEM((SEQ, BPW), jnp.int32)]
          + [pltpu.VMEM((SEQ, LANES), jnp.int32)]
          + [pltpu.VMEM((BPW, HIDDEN), jnp.float32)] * NSLOT
          + [pltpu.VMEM((HIDDEN,), jnp.float32)] * NSLOT
          + [pltpu.SemaphoreType.DMA] * (3 * NSLOT)
      ),
  )
  return run(ids_w, table, pos, flags)


def kernel(input_ids, token_embedding, position_embedding):
  ids32 = input_ids.astype(jnp.int32)
  ids_w = ids32.reshape(NW, BPW, SEQ).transpose(0, 2, 1)
  pos_flat = position_embedding.reshape(SEQ * HIDDEN)
  pos_bits = position_embedding.view(jnp.int32) & jnp.int32(0x7FFFFFFF)
  flags = jnp.any(pos_bits != 0, axis=1).astype(jnp.int32)
  flags = jnp.broadcast_to(flags[:, None], (SEQ, LANES))
  out = _embed(ids_w, token_embedding, pos_flat, flags)
  return out.reshape(SEQ, BATCH, HIDDEN).transpose(1, 0, 2)
